# baseline (speedup 1.0000x reference)
import functools

import jax
import jax.numpy as jnp
from jax import lax
from jax.experimental import pallas as pl
from jax.experimental.pallas import tpu as pltpu
from jax.experimental.pallas import tpu_sc as plsc

B = 16384
D = 16
NC = 2
NS = 16
NW = NC * NS
BPW = B // NW
CHUNK = 128
NCHUNK = BPW // CHUNK


def _make_gather(dtype):
    mesh = plsc.VectorSubcoreMesh(core_axis_name="c", subcore_axis_name="s")

    @functools.partial(
        pl.kernel,
        mesh=mesh,
        out_type=jax.ShapeDtypeStruct((B, D), dtype),
        scratch_types=[
            pltpu.VMEM((NCHUNK, CHUNK), jnp.int32),
            pltpu.VMEM((BPW, D), dtype),
            pltpu.SemaphoreType.DMA,
        ],
        compiler_params=pltpu.CompilerParams(use_tc_tiling_on_sc=False),
    )
    def gather_kernel(idx_hbm, table_hbm, out_hbm, idx_v, rows_v, sem):
        wid = lax.axis_index("s") * NC + lax.axis_index("c")
        base = wid * BPW
        pltpu.sync_copy(idx_hbm.at[wid], idx_v)
        copies = [
            pltpu.async_copy(
                table_hbm.at[idx_v.at[j]],
                rows_v.at[pl.ds(j * CHUNK, CHUNK)],
                sem,
            )
            for j in range(NCHUNK)
        ]
        for c in copies:
            c.wait()
        pltpu.sync_copy(rows_v, out_hbm.at[pl.ds(base, BPW)])

    return gather_kernel


def kernel(inputs, table):
    idx = inputs.astype(jnp.int32).reshape(NW, NCHUNK, CHUNK)
    return _make_gather(table.dtype)(idx, table)


# speedup vs baseline: 1.0012x; 1.0012x over previous
import functools

import jax
import jax.numpy as jnp
from jax import lax
from jax.experimental import pallas as pl
from jax.experimental.pallas import tpu as pltpu
from jax.experimental.pallas import tpu_sc as plsc

B = 16384
D = 16
NC = 2
NS = 16
NW = NC * NS
BPW = B // NW
CHUNK = 128
NCHUNK = BPW // CHUNK


def _make_gather(dtype):
    mesh = plsc.VectorSubcoreMesh(core_axis_name="c", subcore_axis_name="s")

    @functools.partial(
        pl.kernel,
        mesh=mesh,
        out_type=jax.ShapeDtypeStruct((B, D), dtype),
        scratch_types=[
            pltpu.VMEM((BPW,), jnp.int32),
            pltpu.VMEM((BPW, D), dtype),
            pltpu.SemaphoreType.DMA,
        ],
        compiler_params=pltpu.CompilerParams(use_tc_tiling_on_sc=False),
    )
    def gather_kernel(idx_hbm, table_hbm, out_hbm, idx_v, rows_v, sem):
        wid = lax.axis_index("s") * NC + lax.axis_index("c")
        base = wid * BPW
        pltpu.sync_copy(idx_hbm.at[pl.ds(base, BPW)], idx_v)
        copies = [
            pltpu.async_copy(
                table_hbm.at[idx_v.at[pl.ds(j * CHUNK, CHUNK)]],
                rows_v.at[pl.ds(j * CHUNK, CHUNK)],
                sem,
            )
            for j in range(NCHUNK)
        ]
        for c in copies:
            c.wait()
        pltpu.sync_copy(rows_v, out_hbm.at[pl.ds(base, BPW)])

    return gather_kernel


def kernel(inputs, table):
    return _make_gather(table.dtype)(inputs.astype(jnp.int32), table)


# speedup vs baseline: 1.0052x; 1.0040x over previous
import functools

import jax
import jax.numpy as jnp
from jax import lax
from jax.experimental import pallas as pl
from jax.experimental.pallas import tpu as pltpu
from jax.experimental.pallas import tpu_sc as plsc

B = 16384
D = 16
NC = 2
NS = 16
NW = NC * NS
BPW = B // NW
CHUNK = 128
NCHUNK = BPW // CHUNK


def _make_gather(dtype):
    mesh = plsc.VectorSubcoreMesh(core_axis_name="c", subcore_axis_name="s")

    @functools.partial(
        pl.kernel,
        mesh=mesh,
        out_type=jax.ShapeDtypeStruct((B, D), dtype),
        scratch_types=[
            pltpu.VMEM((BPW,), jnp.int32),
            pltpu.VMEM((BPW, D), dtype),
            pltpu.SemaphoreType.DMA,
        ],
        compiler_params=pltpu.CompilerParams(
            use_tc_tiling_on_sc=False, skip_device_barrier=True
        ),
    )
    def gather_kernel(idx_hbm, table_hbm, out_hbm, idx_v, rows_v, sem):
        wid = lax.axis_index("s") * NC + lax.axis_index("c")
        base = wid * BPW
        pltpu.sync_copy(idx_hbm.at[pl.ds(base, BPW)], idx_v)
        copies = [
            pltpu.async_copy(
                table_hbm.at[idx_v.at[pl.ds(j * CHUNK, CHUNK)]],
                rows_v.at[pl.ds(j * CHUNK, CHUNK)],
                sem,
            )
            for j in range(NCHUNK)
        ]
        for c in copies:
            c.wait()
        pltpu.sync_copy(rows_v, out_hbm.at[pl.ds(base, BPW)])

    return gather_kernel


def kernel(inputs, table):
    return _make_gather(table.dtype)(inputs.astype(jnp.int32), table)


# speedup vs baseline: 1.8678x; 1.8581x over previous
import functools

import jax
import jax.numpy as jnp
from jax import lax
from jax.experimental import pallas as pl
from jax.experimental.pallas import tpu as pltpu
from jax.experimental.pallas import tpu_sc as plsc

B = 16384
D = 16
NROWS = 100001
NC = 2
NS = 16
NW = NC * NS
HALF = B // 2
CHUNK = 128
NCHUNK = HALF // CHUNK


def _make_gather(dtype):
    mesh = plsc.VectorSubcoreMesh(core_axis_name="c", subcore_axis_name="s")

    @functools.partial(
        pl.kernel,
        mesh=mesh,
        out_type=jax.ShapeDtypeStruct((D, B), dtype),
        scratch_types=[
            pltpu.VMEM((HALF,), jnp.int32),
            pltpu.VMEM((HALF,), dtype),
            pltpu.SemaphoreType.DMA,
        ],
        compiler_params=pltpu.CompilerParams(use_tc_tiling_on_sc=False),
    )
    def gather_kernel(idx_hbm, tflat_hbm, out_hbm, idx_v, row_v, sem):
        wid = lax.axis_index("s") * NC + lax.axis_index("c")
        d = lax.rem(wid, D)
        half = lax.div(wid, D)
        base = half * HALF
        pltpu.sync_copy(idx_hbm.at[pl.ds(base, HALF)], idx_v)
        dbase = d * NROWS
        def shift(j, carry):
            sl = pl.ds(j * 16, 16)
            idx_v[sl] = idx_v[sl] + dbase
            return carry
        lax.fori_loop(0, HALF // 16, shift, 0, unroll=16)
        copies = [
            pltpu.async_copy(
                tflat_hbm.at[idx_v.at[pl.ds(j * CHUNK, CHUNK)]],
                row_v.at[pl.ds(j * CHUNK, CHUNK)],
                sem,
            )
            for j in range(NCHUNK)
        ]
        for c in copies:
            c.wait()
        pltpu.sync_copy(row_v, out_hbm.at[d, pl.ds(base, HALF)])

    return gather_kernel


def kernel(inputs, table):
    tflat = jnp.swapaxes(table, 0, 1).reshape(-1)
    out_t = _make_gather(table.dtype)(inputs.astype(jnp.int32), tflat)
    return jnp.swapaxes(out_t, 0, 1)


# speedup vs baseline: 1.8849x; 1.0092x over previous
import functools

import jax
import jax.numpy as jnp
from jax import lax
from jax.experimental import pallas as pl
from jax.experimental.pallas import tpu as pltpu
from jax.experimental.pallas import tpu_sc as plsc

B = 16384
D = 16
NROWS = 100001
NC = 2
NS = 16
NW = NC * NS
HALF = B // 2
CHUNK = 512
NCHUNK = HALF // CHUNK


def _make_gather(dtype):
    mesh = plsc.VectorSubcoreMesh(core_axis_name="c", subcore_axis_name="s")

    @functools.partial(
        pl.kernel,
        mesh=mesh,
        out_type=jax.ShapeDtypeStruct((D, B), dtype),
        scratch_types=[
            pltpu.VMEM((HALF,), jnp.int32),
            pltpu.VMEM((HALF,), dtype),
            pltpu.SemaphoreType.DMA,
        ],
        compiler_params=pltpu.CompilerParams(use_tc_tiling_on_sc=False),
    )
    def gather_kernel(idx_hbm, tflat_hbm, out_hbm, idx_v, row_v, sem):
        wid = lax.axis_index("s") * NC + lax.axis_index("c")
        d = lax.rem(wid, D)
        half = lax.div(wid, D)
        base = half * HALF
        pltpu.sync_copy(idx_hbm.at[pl.ds(base, HALF)], idx_v)
        dbase = d * NROWS

        def shift(j, carry):
            sl = pl.ds(j * 16, 16)
            idx_v[sl] = idx_v[sl] + dbase
            return carry

        lax.fori_loop(0, HALF // 16, shift, 0, unroll=16)
        copies = [
            pltpu.async_copy(
                tflat_hbm.at[idx_v.at[pl.ds(j * CHUNK, CHUNK)]],
                row_v.at[pl.ds(j * CHUNK, CHUNK)],
                sem,
            )
            for j in range(NCHUNK)
        ]
        for c in copies:
            c.wait()
        pltpu.sync_copy(row_v, out_hbm.at[d, pl.ds(base, HALF)])

    return gather_kernel


def kernel(inputs, table):
    tflat = jnp.swapaxes(table, 0, 1).reshape(-1)
    out_t = _make_gather(table.dtype)(inputs.astype(jnp.int32), tflat)
    return jnp.swapaxes(out_t, 0, 1)


# speedup vs baseline: 1.8857x; 1.0004x over previous
import functools

import jax
import jax.numpy as jnp
from jax import lax
from jax.experimental import pallas as pl
from jax.experimental.pallas import tpu as pltpu
from jax.experimental.pallas import tpu_sc as plsc

B = 16384
D = 16
NROWS = 100001
NC = 2
NS = 16
NW = NC * NS
HALF = B // 2
CHUNK = 2048
NCHUNK = HALF // CHUNK


def _make_gather(dtype):
    mesh = plsc.VectorSubcoreMesh(core_axis_name="c", subcore_axis_name="s")

    @functools.partial(
        pl.kernel,
        mesh=mesh,
        out_type=jax.ShapeDtypeStruct((D, B), dtype),
        scratch_types=[
            pltpu.VMEM((HALF,), jnp.int32),
            pltpu.VMEM((HALF,), dtype),
            pltpu.SemaphoreType.DMA,
        ],
        compiler_params=pltpu.CompilerParams(use_tc_tiling_on_sc=False),
    )
    def gather_kernel(idx_hbm, tflat_hbm, out_hbm, idx_v, row_v, sem):
        wid = lax.axis_index("s") * NC + lax.axis_index("c")
        d = lax.rem(wid, D)
        half = lax.div(wid, D)
        base = half * HALF
        pltpu.sync_copy(idx_hbm.at[pl.ds(base, HALF)], idx_v)
        dbase = d * NROWS

        def shift(j, carry):
            sl = pl.ds(j * 16, 16)
            idx_v[sl] = idx_v[sl] + dbase
            return carry

        lax.fori_loop(0, HALF // 16, shift, 0, unroll=16)
        copies = [
            pltpu.async_copy(
                tflat_hbm.at[idx_v.at[pl.ds(j * CHUNK, CHUNK)]],
                row_v.at[pl.ds(j * CHUNK, CHUNK)],
                sem,
            )
            for j in range(NCHUNK)
        ]
        for c in copies:
            c.wait()
        pltpu.sync_copy(row_v, out_hbm.at[d, pl.ds(base, HALF)])

    return gather_kernel


def kernel(inputs, table):
    tflat = jnp.swapaxes(table, 0, 1).reshape(-1)
    out_t = _make_gather(table.dtype)(inputs.astype(jnp.int32), tflat)
    return jnp.swapaxes(out_t, 0, 1)


# speedup vs baseline: 1.8873x; 1.0009x over previous
import functools

import jax
import jax.numpy as jnp
from jax import lax
from jax.experimental import pallas as pl
from jax.experimental.pallas import tpu as pltpu
from jax.experimental.pallas import tpu_sc as plsc

B = 16384
D = 16
NROWS = 100001
NC = 2
NS = 16
NW = NC * NS
HALF = B // 2
CHUNK = 8192
NCHUNK = HALF // CHUNK


def _make_gather(dtype):
    mesh = plsc.VectorSubcoreMesh(core_axis_name="c", subcore_axis_name="s")

    @functools.partial(
        pl.kernel,
        mesh=mesh,
        out_type=jax.ShapeDtypeStruct((D, B), dtype),
        scratch_types=[
            pltpu.VMEM((HALF,), jnp.int32),
            pltpu.VMEM((HALF,), dtype),
            pltpu.SemaphoreType.DMA,
        ],
        compiler_params=pltpu.CompilerParams(use_tc_tiling_on_sc=False),
    )
    def gather_kernel(idx_hbm, tflat_hbm, out_hbm, idx_v, row_v, sem):
        wid = lax.axis_index("s") * NC + lax.axis_index("c")
        d = lax.rem(wid, D)
        half = lax.div(wid, D)
        base = half * HALF
        pltpu.sync_copy(idx_hbm.at[pl.ds(base, HALF)], idx_v)
        dbase = d * NROWS

        def shift(j, carry):
            sl = pl.ds(j * 16, 16)
            idx_v[sl] = idx_v[sl] + dbase
            return carry

        lax.fori_loop(0, HALF // 16, shift, 0, unroll=16)
        copies = [
            pltpu.async_copy(
                tflat_hbm.at[idx_v.at[pl.ds(j * CHUNK, CHUNK)]],
                row_v.at[pl.ds(j * CHUNK, CHUNK)],
                sem,
            )
            for j in range(NCHUNK)
        ]
        for c in copies:
            c.wait()
        pltpu.sync_copy(row_v, out_hbm.at[d, pl.ds(base, HALF)])

    return gather_kernel


def kernel(inputs, table):
    tflat = jnp.swapaxes(table, 0, 1).reshape(-1)
    out_t = _make_gather(table.dtype)(inputs.astype(jnp.int32), tflat)
    return jnp.swapaxes(out_t, 0, 1)


# speedup vs baseline: 1.8907x; 1.0018x over previous
import functools

import jax
import jax.numpy as jnp
from jax import lax
from jax.experimental import pallas as pl
from jax.experimental.pallas import tpu as pltpu
from jax.experimental.pallas import tpu_sc as plsc

B = 16384
D = 16
NROWS = 100001
NC = 2
NS = 16
NW = NC * NS
HALF = B // 2
CHUNK = 2048
NCHUNK = HALF // CHUNK


def _make_gather(dtype):
    mesh = plsc.VectorSubcoreMesh(core_axis_name="c", subcore_axis_name="s")

    @functools.partial(
        pl.kernel,
        mesh=mesh,
        out_type=jax.ShapeDtypeStruct((D, B), dtype),
        scratch_types=[
            pltpu.VMEM((HALF,), jnp.int32),
            pltpu.VMEM((HALF,), dtype),
            pltpu.SemaphoreType.DMA,
        ],
        compiler_params=pltpu.CompilerParams(use_tc_tiling_on_sc=False),
    )
    def gather_kernel(idx_hbm, tflat_hbm, out_hbm, idx_v, row_v, sem):
        wid = lax.axis_index("s") * NC + lax.axis_index("c")
        d = lax.rem(wid, D)
        half = lax.div(wid, D)
        base = half * HALF
        pltpu.sync_copy(idx_hbm.at[pl.ds(base, HALF)], idx_v)
        dbase = d * NROWS

        def shift(j, carry):
            sl = pl.ds(j * 16, 16)
            idx_v[sl] = idx_v[sl] + dbase
            return carry

        lax.fori_loop(0, HALF // 16, shift, 0, unroll=16)
        copies = [
            pltpu.async_copy(
                tflat_hbm.at[idx_v.at[pl.ds(j * CHUNK, CHUNK)]],
                row_v.at[pl.ds(j * CHUNK, CHUNK)],
                sem,
            )
            for j in range(NCHUNK)
        ]
        for c in copies:
            c.wait()
        pltpu.sync_copy(row_v, out_hbm.at[d, pl.ds(base, HALF)])

    return gather_kernel


def kernel(inputs, table):
    tflat = jnp.swapaxes(table, 0, 1).reshape(-1)
    out_t = _make_gather(table.dtype)(inputs.astype(jnp.int32), tflat)
    return jnp.swapaxes(out_t, 0, 1)
